# baseline (speedup 1.0000x reference)
import functools
import jax
import jax.numpy as jnp
from jax import lax
from jax.experimental import pallas as pl
from jax.experimental.pallas import tpu as pltpu
from jax.experimental.pallas import tpu_sc as plsc

RESO = 128
L = 16
NC = 2
NS = 16
NW = NC * NS
CHUNK = 4096
GROUPS = CHUNK // L


def _quadric_grid_sc(P):
    n_chunks = -(-P // CHUNK)
    s_max = -(-n_chunks // NW)
    last_base = P - CHUNK
    mesh = plsc.VectorSubcoreMesh(core_axis_name="c", subcore_axis_name="s",
                                  num_cores=NC, num_subcores=NS)

    @functools.partial(
        pl.kernel,
        out_type=(
            jax.ShapeDtypeStruct((P,), jnp.float32),
            jax.ShapeDtypeStruct((3 * P,), jnp.float32),
        ),
        mesh=mesh,
        compiler_params=pltpu.CompilerParams(needs_layout_passes=False),
        scratch_types=dict(
            xl=pltpu.VMEM((RESO,), jnp.float32),
            yl=pltpu.VMEM((RESO,), jnp.float32),
            zl=pltpu.VMEM((RESO,), jnp.float32),
            off=pltpu.VMEM((4 * L,), jnp.float32),
            idx_v=pltpu.VMEM((CHUNK,), jnp.int32),
            pts_v=pltpu.VMEM((3 * CHUNK,), jnp.float32),
            sdf_v=pltpu.VMEM((CHUNK,), jnp.float32),
            nrm_v=pltpu.VMEM((3 * CHUNK,), jnp.float32),
        ),
    )
    def k(rpts_hbm, ridx_hbm, spts_hbm, sidx_hbm, xl_hbm, yl_hbm, zl_hbm,
          off_hbm, sdf_out_hbm, nrm_out_hbm,
          xl, yl, zl, off, idx_v, pts_v, sdf_v, nrm_v):
        wid = lax.axis_index("s") * NC + lax.axis_index("c")

        pltpu.sync_copy(xl_hbm, xl)
        pltpu.sync_copy(yl_hbm, yl)
        pltpu.sync_copy(zl_hbm, zl)
        pltpu.sync_copy(off_hbm, off)

        d = off[pl.ds(0 * L, L)]
        e = off[pl.ds(1 * L, L)]
        f = off[pl.ds(2 * L, L)]
        g = off[pl.ds(3 * L, L)]
        lane = lax.iota(jnp.int32, L)
        lane3 = lane * 3

        def compute_chunk(want_normal):
            @pl.loop(0, GROUPS)
            def _(grp):
                base16 = grp * L
                idx = idx_v[pl.ds(base16, L)]
                ii = lax.shift_right_logical(idx, 14)
                jj = lax.shift_right_logical(idx, 7) & (RESO - 1)
                kk = idx & (RESO - 1)
                a = plsc.load_gather(xl, [ii])
                b = plsc.load_gather(yl, [jj])
                c = plsc.load_gather(zl, [kk])
                p3 = base16 * 3 + lane3
                x = plsc.load_gather(pts_v, [p3])
                y = plsc.load_gather(pts_v, [p3 + 1])
                z = plsc.load_gather(pts_v, [p3 + 2])
                if want_normal:
                    nx = (a + a) * x + d
                    ny = (b + b) * y + e
                    nz = (c + c) * z + f
                    plsc.store_scatter(nrm_v, [p3], nx)
                    plsc.store_scatter(nrm_v, [p3 + 1], ny)
                    plsc.store_scatter(nrm_v, [p3 + 2], nz)
                else:
                    sdf = (a * x * x + b * y * y + c * z * z
                           + d * x + e * y + f * z + g)
                    sdf_v[pl.ds(base16, L)] = sdf

        for s in range(s_max):
            t = wid + s * NW

            @pl.when(t < n_chunks)
            def _():
                base = jnp.minimum(t * CHUNK, last_base)
                pltpu.sync_copy(sidx_hbm.at[pl.ds(base, CHUNK)], idx_v)
                pltpu.sync_copy(spts_hbm.at[pl.ds(base * 3, 3 * CHUNK)], pts_v)
                compute_chunk(False)
                pltpu.sync_copy(sdf_v, sdf_out_hbm.at[pl.ds(base, CHUNK)])
                pltpu.sync_copy(ridx_hbm.at[pl.ds(base, CHUNK)], idx_v)
                pltpu.sync_copy(rpts_hbm.at[pl.ds(base * 3, 3 * CHUNK)], pts_v)
                compute_chunk(True)
                pltpu.sync_copy(nrm_v, nrm_out_hbm.at[pl.ds(base * 3, 3 * CHUNK)])

    return k


@jax.jit
def kernel(renderPointList, renderIndexList, sdfPointList, sdfIndexList,
           xLayer, yLayer, zLayer, offset):
    P = renderPointList.shape[0]
    k = _quadric_grid_sc(P)
    off64 = jnp.repeat(offset, L)
    sdf, nrm = k(renderPointList.reshape(-1), renderIndexList,
                 sdfPointList.reshape(-1), sdfIndexList,
                 xLayer, yLayer, zLayer, off64)
    return sdf, nrm.reshape(P, 3)
